# baseline (speedup 1.0000x reference)
import jax
import jax.numpy as jnp
from jax import lax
from jax.experimental import pallas as pl
from jax.experimental.pallas import tpu as pltpu
from jax.experimental.pallas import tpu_sc as plsc

_B, _C, _H, _W = 32, 96, 224, 224
_CCHUNK = 48
_NC = _C // _CCHUNK
_TOPKS = (3, 4, 5, 6, 7, 10, 15, 20)
_NV = 7
_PAD = -1e38


def _tc_kernel(in_ref, w1_ref, w2_ref, x_sun_ref, votes_ref, copy_ref,
               acc_ref):
    c = pl.program_id(0)
    b = pl.program_id(1)
    x = in_ref[...]
    copy_ref[...] = x
    rows = jnp.sum(x, axis=2)[0]

    for ci in range(_NC):
        lo = ci * _CCHUNK
        if lo >= 40:
            break
        n = min(_CCHUNK, 40 - lo)

        @pl.when(c == ci)
        def _(lo=lo, n=n):
            acc_ref[b, lo:lo + n, :] = rows[0:n, :]

    @pl.when(jnp.logical_and(c == 0, b == _B - 1))
    def _():
        p = acc_ref[:, :33, :]
        sums = jnp.sum(p, axis=2)
        x_sun = sums * (1.0 / (_H * _W))
        x_sun_ref[...] = x_sun

        xt = jnp.tanh(x_sun)
        xg = jax.lax.dot_general(
            xt, w1_ref[...], (((1,), (1,)), ((), ())),
            preferred_element_type=jnp.float32)
        xg = jnp.maximum(xg, 0.0)

        votes = xg[:, None, :] * w2_ref[...][None, :, :]
        votes_ref[:, :, 0:100] = votes
        votes_ref[:, :, 100:112] = jnp.full((_B, 10, 12), _PAD, jnp.float32)


def _sc_topk_kernel(votes_hbm, out_hbm, votes_v, out_v):
    wid = lax.axis_index("s") * 2 + lax.axis_index("c")
    pltpu.sync_copy(votes_hbm.at[wid], votes_v)

    lane = lax.iota(jnp.int32, 16)
    kslot = {k: i for i, k in enumerate(_TOPKS)}
    shuf = [(lane ^ s).astype(jnp.int32) for s in (1, 2, 4, 8)]

    def _bcast_max(v):
        for idx in shuf:
            v = jnp.maximum(v, v.at[idx].get(mode="promise_in_bounds"))
        return v

    def _bcast_sum(v):
        for idx in shuf:
            v = v + v.at[idx].get(mode="promise_in_bounds")
        return v

    def row_body(r, carry):
        vs = [votes_v[r, pl.ds(16 * j, 16)] for j in range(_NV)]
        s = vs[0] + vs[1] + vs[2] + vs[3] + vs[4] + vs[5]
        s = s + jnp.where(lane < 4, vs[6], jnp.float32(0.0))
        linear = _bcast_sum(s)
        ks = [
            lax.bitcast_convert_type(
                (lax.bitcast_convert_type(vs[j], jnp.int32)
                 & jnp.int32(-128)) | (lane + 16 * j),
                jnp.float32)
            for j in range(_NV)
        ]
        recs = []
        acc = jnp.zeros((16,), jnp.float32)
        for i in range(1, max(_TOPKS) + 1):
            mv = ks[0]
            for j in range(1, _NV):
                mv = jnp.maximum(mv, ks[j])
            m = _bcast_max(mv)
            acc = acc + m
            if i in kslot:
                recs.append(acc)
            ks = [jnp.where(kj == m, jnp.float32(_PAD * 4), kj) for kj in ks]
        out = jnp.zeros((16,), jnp.float32)
        for slot, val in enumerate(recs + [linear]):
            out = jnp.where(lane == slot, val, out)
        out_v[r, :] = out
        return carry

    lax.fori_loop(0, 10, row_body, 0)
    pltpu.sync_copy(out_v, out_hbm.at[wid])


def kernel(maps, W1, W2):
    x_sun, votes_p, maps_copy = pl.pallas_call(
        _tc_kernel,
        grid=(_NC, _B),
        in_specs=[pl.BlockSpec((1, _CCHUNK, _H, _W),
                               lambda c, b: (b, c, 0, 0)),
                  pl.BlockSpec(W1.shape, lambda c, b: (0, 0)),
                  pl.BlockSpec(W2.shape, lambda c, b: (0, 0))],
        out_specs=[pl.BlockSpec((_B, 33), lambda c, b: (0, 0)),
                   pl.BlockSpec((_B, 10, 16 * _NV), lambda c, b: (0, 0, 0)),
                   pl.BlockSpec((1, _CCHUNK, _H, _W),
                                lambda c, b: (b, c, 0, 0))],
        out_shape=[jax.ShapeDtypeStruct((_B, 33), jnp.float32),
                   jax.ShapeDtypeStruct((_B, 10, 16 * _NV), jnp.float32),
                   jax.ShapeDtypeStruct((_B, _C, _H, _W), jnp.float32)],
        scratch_shapes=[pltpu.VMEM((_B, 40, _W), jnp.float32)],
        compiler_params=pltpu.CompilerParams(
            dimension_semantics=("arbitrary", "arbitrary")),
    )(maps, W1, W2)

    raw = pl.kernel(
        _sc_topk_kernel,
        mesh=plsc.VectorSubcoreMesh(core_axis_name="c", subcore_axis_name="s"),
        out_type=jax.ShapeDtypeStruct((_B, 10, 16), jnp.float32),
        scratch_types=[pltpu.VMEM((10, 16 * _NV), jnp.float32),
                       pltpu.VMEM((10, 16), jnp.float32)],
    )(votes_p)

    x_son = jnp.transpose(raw[:, :, :9], (2, 0, 1))
    return (x_sun, x_son, maps_copy)


# speedup vs baseline: 1.0110x; 1.0110x over previous
import jax
import jax.numpy as jnp
from jax import lax
from jax.experimental import pallas as pl
from jax.experimental.pallas import tpu as pltpu
from jax.experimental.pallas import tpu_sc as plsc

_B, _C, _H, _W = 32, 96, 224, 224
_CCHUNK = 48
_NC = _C // _CCHUNK
_TOPKS = (3, 4, 5, 6, 7, 10, 15, 20)
_NV = 7
_PAD = -1e38


def _tc_stream_prep(in_ref, w1_ref, w2_ref, x_sun_ref, votes_ref, copy_ref,
                    acc_ref):
    b = pl.program_id(0)
    x = in_ref[...]
    copy_ref[...] = x
    rows = jnp.sum(x, axis=2)[0]
    acc_ref[b, 0:40, :] = rows[0:40, :]

    @pl.when(b == _B - 1)
    def _():
        p = acc_ref[:, :33, :]
        sums = jnp.sum(p, axis=2)
        x_sun = sums * (1.0 / (_H * _W))
        x_sun_ref[...] = x_sun

        xt = jnp.tanh(x_sun)
        xg = jax.lax.dot_general(
            xt, w1_ref[...], (((1,), (1,)), ((), ())),
            preferred_element_type=jnp.float32)
        xg = jnp.maximum(xg, 0.0)

        votes = xg[:, None, :] * w2_ref[...][None, :, :]
        votes_ref[:, :, 0:100] = votes
        votes_ref[:, :, 100:112] = jnp.full((_B, 10, 12), _PAD, jnp.float32)


def _sc_topk_kernel(votes_hbm, out_hbm, votes_v, out_v):
    wid = lax.axis_index("s") * 2 + lax.axis_index("c")
    pltpu.sync_copy(votes_hbm.at[wid], votes_v)

    lane = lax.iota(jnp.int32, 16)
    kslot = {k: i for i, k in enumerate(_TOPKS)}
    shuf = [(lane ^ s).astype(jnp.int32) for s in (1, 2, 4, 8)]

    def _bcast_max(v):
        for idx in shuf:
            v = jnp.maximum(v, v.at[idx].get(mode="promise_in_bounds"))
        return v

    def _bcast_sum(v):
        for idx in shuf:
            v = v + v.at[idx].get(mode="promise_in_bounds")
        return v

    def row_body(r, carry):
        vs = [votes_v[r, pl.ds(16 * j, 16)] for j in range(_NV)]
        s = vs[0] + vs[1] + vs[2] + vs[3] + vs[4] + vs[5]
        s = s + jnp.where(lane < 4, vs[6], jnp.float32(0.0))
        linear = _bcast_sum(s)
        ks = [
            lax.bitcast_convert_type(
                (lax.bitcast_convert_type(vs[j], jnp.int32)
                 & jnp.int32(-128)) | (lane + 16 * j),
                jnp.float32)
            for j in range(_NV)
        ]
        recs = []
        acc = jnp.zeros((16,), jnp.float32)
        for i in range(1, max(_TOPKS) + 1):
            mv = ks[0]
            for j in range(1, _NV):
                mv = jnp.maximum(mv, ks[j])
            m = _bcast_max(mv)
            acc = acc + m
            if i in kslot:
                recs.append(acc)
            ks = [jnp.where(kj == m, jnp.float32(_PAD * 4), kj) for kj in ks]
        out = jnp.zeros((16,), jnp.float32)
        for slot, val in enumerate(recs + [linear]):
            out = jnp.where(lane == slot, val, out)
        out_v[r, :] = out
        return carry

    lax.fori_loop(0, 10, row_body, 0)
    pltpu.sync_copy(out_v, out_hbm.at[wid])


def _tc_copy_rest(prev_ref, in_ref, out_ref):
    del prev_ref
    out_ref[...] = in_ref[...]


def kernel(maps, W1, W2):
    x_sun, votes_p, copy_half = pl.pallas_call(
        _tc_stream_prep,
        grid=(_B,),
        in_specs=[pl.BlockSpec((1, _CCHUNK, _H, _W),
                               lambda b: (b, 0, 0, 0)),
                  pl.BlockSpec(W1.shape, lambda b: (0, 0)),
                  pl.BlockSpec(W2.shape, lambda b: (0, 0))],
        out_specs=[pl.BlockSpec((_B, 33), lambda b: (0, 0)),
                   pl.BlockSpec((_B, 10, 16 * _NV), lambda b: (0, 0, 0)),
                   pl.BlockSpec((1, _CCHUNK, _H, _W),
                                lambda b: (b, 0, 0, 0))],
        out_shape=[jax.ShapeDtypeStruct((_B, 33), jnp.float32),
                   jax.ShapeDtypeStruct((_B, 10, 16 * _NV), jnp.float32),
                   jax.ShapeDtypeStruct((_B, _C, _H, _W), jnp.float32)],
        scratch_shapes=[pltpu.VMEM((_B, 40, _W), jnp.float32)],
        compiler_params=pltpu.CompilerParams(
            dimension_semantics=("arbitrary",)),
    )(maps, W1, W2)

    maps_copy = pl.pallas_call(
        _tc_copy_rest,
        grid=(_B,),
        in_specs=[pl.BlockSpec(memory_space=pl.ANY),
                  pl.BlockSpec((1, _CCHUNK, _H, _W),
                               lambda b: (b, 1, 0, 0))],
        out_specs=pl.BlockSpec((1, _CCHUNK, _H, _W),
                               lambda b: (b, 1, 0, 0)),
        out_shape=jax.ShapeDtypeStruct((_B, _C, _H, _W), jnp.float32),
        input_output_aliases={0: 0},
        compiler_params=pltpu.CompilerParams(
            dimension_semantics=("arbitrary",)),
    )(copy_half, maps)

    raw = pl.kernel(
        _sc_topk_kernel,
        mesh=plsc.VectorSubcoreMesh(core_axis_name="c", subcore_axis_name="s"),
        out_type=jax.ShapeDtypeStruct((_B, 10, 16), jnp.float32),
        scratch_types=[pltpu.VMEM((10, 16 * _NV), jnp.float32),
                       pltpu.VMEM((10, 16), jnp.float32)],
    )(votes_p)

    x_son = jnp.transpose(raw[:, :, :9], (2, 0, 1))
    return (x_sun, x_son, maps_copy)
